# baseline (speedup 1.0000x reference)
import functools

import jax
import jax.numpy as jnp
from jax import lax
from jax.experimental import pallas as pl
from jax.experimental.pallas import tpu as pltpu
from jax.experimental.pallas import tpu_sc as plsc

_T = 1000000
_N = 4096
_L = 16
_C = 4096
_R = _C // 128
_NW = 32


def _p_init_sc(cols_hbm, a_flat, t):
    per_w = (t // _NW) // 16 * 16
    last_span = t - (_NW - 1) * per_w
    nchunk = -(-max(per_w, last_span) // _C)
    mesh = plsc.VectorSubcoreMesh(core_axis_name="c", subcore_axis_name="s")

    buf_set = [
        pltpu.VMEM((_C,), jnp.int32),
        pltpu.VMEM((_C,), jnp.int32),
        pltpu.VMEM((_C,), jnp.int32),
        pltpu.VMEM((3 * _C,), jnp.int32),
        pltpu.VMEM((3 * _C,), jnp.float32),
        pltpu.VMEM((_C,), jnp.float32),
    ]

    @functools.partial(
        pl.kernel,
        out_type=jax.ShapeDtypeStruct((t,), jnp.float32),
        mesh=mesh,
        scratch_types=buf_set + buf_set + [
            pltpu.SemaphoreType.DMA,
            pltpu.SemaphoreType.DMA,
            pltpu.SemaphoreType.DMA,
            pltpu.SemaphoreType.DMA,
            pltpu.SemaphoreType.DMA,
            pltpu.SemaphoreType.DMA,
        ],
        compiler_params=pltpu.CompilerParams(needs_layout_passes=False),
    )
    def k(ch, a_hbm, out_hbm, *scratch):
        bufs = (scratch[0:6], scratch[6:12])
        sems = scratch[12:14]
        isems = scratch[14:16]
        osems = scratch[16:18]
        nc = 2
        wid = lax.axis_index("s") * nc + lax.axis_index("c")
        start = wid * per_w
        end = jnp.where(wid == _NW - 1, t, start + per_w)
        lastbase = end - _C

        def chunk_base(c):
            return jnp.minimum(start + c * _C, lastbase)

        def idx_start(c, bset, sem):
            bi, bj, bk = bset[0:3]
            base = chunk_base(c)
            return [
                pltpu.async_copy(ch.at[pl.ds(col * t + base, _C)], dst, sem)
                for col, dst in ((0, bi), (1, bj), (2, bk))
            ]

        def offsets(bset):
            bi, bj, bk, f = bset[0:4]

            def fbody(r, carry):
                for g in range(8):
                    off = r * 128 + g * 16
                    iv = bi[pl.ds(off, 16)]
                    jv = bj[pl.ds(off, 16)]
                    kv = bk[pl.ds(off, 16)]
                    rpi = lax.shift_left(iv & 0xFF8, 12) | lax.shift_left(iv & 7, 7)
                    rpj = lax.shift_left(jv & 0xFF8, 12) | lax.shift_left(jv & 7, 7)
                    cpj = lax.shift_left(jv & 0xF80, 3) | (jv & 0x7F)
                    cpk = lax.shift_left(kv & 0xF80, 3) | (kv & 0x7F)
                    f[pl.ds(off, 16)] = rpi + cpj
                    f[pl.ds(_C + off, 16)] = rpi + cpk
                    f[pl.ds(2 * _C + off, 16)] = rpj + cpk
                return carry

            lax.fori_loop(0, _R, fbody, 0)

        def gather_start(bset, sem):
            f, g = bset[3:5]
            return [pltpu.async_copy(a_hbm.at[f], g, sem)]

        def combine(c, bset, sem):
            g, obuf = bset[4:6]
            base = chunk_base(c)

            def obody(r, carry):
                for gi in range(8):
                    off = r * 128 + gi * 16
                    s = (g[pl.ds(off, 16)]
                         + g[pl.ds(_C + off, 16)]
                         + g[pl.ds(2 * _C + off, 16)])
                    obuf[pl.ds(off, 16)] = s * (1.0 / 3.0)
                return carry

            lax.fori_loop(0, _R, obody, 0)
            return pltpu.async_copy(obuf, out_hbm.at[pl.ds(base, _C)], sem)

        def wait_all(copies):
            for cp in copies:
                cp.wait()

        idx_inflight = idx_start(0, bufs[0], isems[0])
        wait_all(idx_inflight)
        offsets(bufs[0])
        g_inflight = [None, None]
        out_inflight = [None, None]
        g_inflight[0] = gather_start(bufs[0], sems[0])
        idx_inflight = idx_start(1, bufs[1], isems[1])
        for c in range(1, nchunk):
            s = c % 2
            wait_all(idx_inflight)
            offsets(bufs[s])
            g_inflight[s] = gather_start(bufs[s], sems[s])
            if c + 1 < nchunk:
                idx_inflight = idx_start(c + 1, bufs[1 - s], isems[1 - s])
            wait_all(g_inflight[1 - s])
            if out_inflight[1 - s] is not None:
                out_inflight[1 - s].wait()
            out_inflight[1 - s] = combine(c - 1, bufs[1 - s], osems[1 - s])
        s = (nchunk - 1) % 2
        wait_all(g_inflight[s])
        if out_inflight[s] is not None:
            out_inflight[s].wait()
        combine(nchunk - 1, bufs[s], osems[s]).wait()
        if out_inflight[1 - s] is not None:
            out_inflight[1 - s].wait()

    return k(cols_hbm, a_flat)


_PARTS = (139264, 360448, 500288)


def kernel(triangles_indexes, A_s, triangles):
    del triangles
    tidx = triangles_indexes.astype(jnp.int32)
    table = A_s.reshape(512, 8, 32, 128).transpose(0, 2, 1, 3).reshape(-1)
    outs = []
    lo = 0
    for h in _PARTS:
        cols = tidx[lo:lo + h].T.reshape(-1)
        outs.append(_p_init_sc(cols, table, h))
        lo += h
    return jnp.concatenate(outs)


# speedup vs baseline: 1.1217x; 1.1217x over previous
import functools

import jax
import jax.numpy as jnp
from jax import lax
from jax.experimental import pallas as pl
from jax.experimental.pallas import tpu as pltpu
from jax.experimental.pallas import tpu_sc as plsc

_T = 1000000
_N = 4096
_L = 16
_C = 4096
_R = _C // 128
_NW = 32


def _p_init_sc(cols_hbm, a_flat, t):
    per_w = (t // _NW) // 16 * 16
    last_span = t - (_NW - 1) * per_w
    nchunk = -(-max(per_w, last_span) // _C)
    mesh = plsc.VectorSubcoreMesh(core_axis_name="c", subcore_axis_name="s")

    buf_set = [
        pltpu.VMEM((_C,), jnp.int32),
        pltpu.VMEM((_C,), jnp.int32),
        pltpu.VMEM((_C,), jnp.int32),
        pltpu.VMEM((3 * _C,), jnp.int32),
        pltpu.VMEM((3 * _C,), jnp.float32),
        pltpu.VMEM((_C,), jnp.float32),
    ]

    @functools.partial(
        pl.kernel,
        out_type=jax.ShapeDtypeStruct((t,), jnp.float32),
        mesh=mesh,
        scratch_types=buf_set + buf_set + [
            pltpu.SemaphoreType.DMA,
            pltpu.SemaphoreType.DMA,
            pltpu.SemaphoreType.DMA,
            pltpu.SemaphoreType.DMA,
            pltpu.SemaphoreType.DMA,
            pltpu.SemaphoreType.DMA,
        ],
        compiler_params=pltpu.CompilerParams(needs_layout_passes=False),
    )
    def k(ch, a_hbm, out_hbm, *scratch):
        bufs = (scratch[0:6], scratch[6:12])
        sems = scratch[12:14]
        isems = scratch[14:16]
        osems = scratch[16:18]
        nc = 2
        wid = lax.axis_index("s") * nc + lax.axis_index("c")
        start = wid * per_w
        end = jnp.where(wid == _NW - 1, t, start + per_w)
        lastbase = end - _C

        def chunk_base(c):
            return jnp.minimum(start + c * _C, lastbase)

        def idx_start(c, bset, sem):
            bi, bj, bk = bset[0:3]
            base = chunk_base(c)
            return [
                pltpu.async_copy(ch.at[pl.ds(col * t + base, _C)], dst, sem)
                for col, dst in ((0, bi), (1, bj), (2, bk))
            ]

        def offsets(bset):
            bi, bj, bk, f = bset[0:4]

            def fbody(r, carry):
                for g in range(8):
                    off = r * 128 + g * 16
                    iv = bi[pl.ds(off, 16)]
                    jv = bj[pl.ds(off, 16)]
                    kv = bk[pl.ds(off, 16)]
                    rpi = lax.shift_left(iv & 0xFF8, 12) | lax.shift_left(iv & 7, 7)
                    rpj = lax.shift_left(jv & 0xFF8, 12) | lax.shift_left(jv & 7, 7)
                    cpj = lax.shift_left(jv & 0xF80, 3) | (jv & 0x7F)
                    cpk = lax.shift_left(kv & 0xF80, 3) | (kv & 0x7F)
                    f[pl.ds(off, 16)] = rpi + cpj
                    f[pl.ds(_C + off, 16)] = rpi + cpk
                    f[pl.ds(2 * _C + off, 16)] = rpj + cpk
                return carry

            lax.fori_loop(0, _R, fbody, 0)

        def gather_start(bset, sem):
            f, g = bset[3:5]
            return [pltpu.async_copy(a_hbm.at[f], g, sem)]

        def combine(c, bset, sem):
            g, obuf = bset[4:6]
            base = chunk_base(c)

            def obody(r, carry):
                for gi in range(8):
                    off = r * 128 + gi * 16
                    s = (g[pl.ds(off, 16)]
                         + g[pl.ds(_C + off, 16)]
                         + g[pl.ds(2 * _C + off, 16)])
                    obuf[pl.ds(off, 16)] = s * (1.0 / 3.0)
                return carry

            lax.fori_loop(0, _R, obody, 0)
            return pltpu.async_copy(obuf, out_hbm.at[pl.ds(base, _C)], sem)

        def wait_all(copies):
            for cp in copies:
                cp.wait()

        idx_inflight = idx_start(0, bufs[0], isems[0])
        wait_all(idx_inflight)
        offsets(bufs[0])
        g_inflight = [None, None]
        out_inflight = [None, None]
        g_inflight[0] = gather_start(bufs[0], sems[0])
        idx_inflight = idx_start(1, bufs[1], isems[1])
        for c in range(1, nchunk):
            s = c % 2
            wait_all(idx_inflight)
            offsets(bufs[s])
            g_inflight[s] = gather_start(bufs[s], sems[s])
            if c + 1 < nchunk:
                idx_inflight = idx_start(c + 1, bufs[1 - s], isems[1 - s])
            wait_all(g_inflight[1 - s])
            if out_inflight[1 - s] is not None:
                out_inflight[1 - s].wait()
            out_inflight[1 - s] = combine(c - 1, bufs[1 - s], osems[1 - s])
        s = (nchunk - 1) % 2
        wait_all(g_inflight[s])
        if out_inflight[s] is not None:
            out_inflight[s].wait()
        combine(nchunk - 1, bufs[s], osems[s]).wait()
        if out_inflight[1 - s] is not None:
            out_inflight[1 - s].wait()

    return k(cols_hbm, a_flat)


_PARTS = (499968, 500032)


def kernel(triangles_indexes, A_s, triangles):
    del triangles
    tidx = triangles_indexes.astype(jnp.int32)
    table = A_s.reshape(512, 8, 32, 128).transpose(0, 2, 1, 3).reshape(-1)
    outs = []
    lo = 0
    for h in _PARTS:
        cols = tidx[lo:lo + h].T.reshape(-1)
        outs.append(_p_init_sc(cols, table, h))
        lo += h
    return jnp.concatenate(outs)


# speedup vs baseline: 1.1298x; 1.0072x over previous
import functools

import jax
import jax.numpy as jnp
from jax import lax
from jax.experimental import pallas as pl
from jax.experimental.pallas import tpu as pltpu
from jax.experimental.pallas import tpu_sc as plsc

_T = 1000000
_N = 4096
_L = 16
_C = 4096
_R = _C // 128
_NW = 32


def _p_init_sc(cols_hbm, a_flat, t):
    per_w = (t // _NW) // 16 * 16
    last_span = t - (_NW - 1) * per_w
    nchunk = -(-max(per_w, last_span) // _C)
    mesh = plsc.VectorSubcoreMesh(core_axis_name="c", subcore_axis_name="s")

    buf_set = [
        pltpu.VMEM((_C,), jnp.int32),
        pltpu.VMEM((_C,), jnp.int32),
        pltpu.VMEM((_C,), jnp.int32),
        pltpu.VMEM((3 * _C,), jnp.int32),
        pltpu.VMEM((3 * _C,), jnp.float32),
        pltpu.VMEM((_C,), jnp.float32),
    ]

    @functools.partial(
        pl.kernel,
        out_type=jax.ShapeDtypeStruct((t,), jnp.float32),
        mesh=mesh,
        scratch_types=buf_set + buf_set + [
            pltpu.SemaphoreType.DMA,
            pltpu.SemaphoreType.DMA,
            pltpu.SemaphoreType.DMA,
            pltpu.SemaphoreType.DMA,
            pltpu.SemaphoreType.DMA,
            pltpu.SemaphoreType.DMA,
        ],
        compiler_params=pltpu.CompilerParams(needs_layout_passes=False),
    )
    def k(ch, a_hbm, out_hbm, *scratch):
        bufs = (scratch[0:6], scratch[6:12])
        sems = scratch[12:14]
        isems = scratch[14:16]
        osems = scratch[16:18]
        nc = 2
        wid = lax.axis_index("s") * nc + lax.axis_index("c")
        start = wid * per_w
        end = jnp.where(wid == _NW - 1, t, start + per_w)
        lastbase = end - _C

        def chunk_base(c):
            return jnp.minimum(start + c * _C, lastbase)

        def idx_start(c, bset, sem):
            bi, bj, bk = bset[0:3]
            base = chunk_base(c)
            return [
                pltpu.async_copy(ch.at[pl.ds(col * t + base, _C)], dst, sem)
                for col, dst in ((0, bi), (1, bj), (2, bk))
            ]

        def offsets(bset):
            bi, bj, bk, f = bset[0:4]

            def fbody(r, carry):
                for g in range(8):
                    off = r * 128 + g * 16
                    iv = bi[pl.ds(off, 16)]
                    jv = bj[pl.ds(off, 16)]
                    kv = bk[pl.ds(off, 16)]
                    rpi = lax.shift_left(iv & 0xFF8, 12) | lax.shift_left(iv & 7, 7)
                    rpj = lax.shift_left(jv & 0xFF8, 12) | lax.shift_left(jv & 7, 7)
                    cpj = lax.shift_left(jv & 0xF80, 3) | (jv & 0x7F)
                    cpk = lax.shift_left(kv & 0xF80, 3) | (kv & 0x7F)
                    f[pl.ds(off, 16)] = rpi + cpj
                    f[pl.ds(_C + off, 16)] = rpi + cpk
                    f[pl.ds(2 * _C + off, 16)] = rpj + cpk
                return carry

            lax.fori_loop(0, _R, fbody, 0)

        def gather_start(bset, sem):
            f, g = bset[3:5]
            return [pltpu.async_copy(a_hbm.at[f], g, sem)]

        def combine(c, bset, sem):
            g, obuf = bset[4:6]
            base = chunk_base(c)

            def obody(r, carry):
                for gi in range(8):
                    off = r * 128 + gi * 16
                    s = (g[pl.ds(off, 16)]
                         + g[pl.ds(_C + off, 16)]
                         + g[pl.ds(2 * _C + off, 16)])
                    obuf[pl.ds(off, 16)] = s * (1.0 / 3.0)
                return carry

            lax.fori_loop(0, _R, obody, 0)
            return pltpu.async_copy(obuf, out_hbm.at[pl.ds(base, _C)], sem)

        def wait_all(copies):
            for cp in copies:
                cp.wait()

        idx_inflight = idx_start(0, bufs[0], isems[0])
        wait_all(idx_inflight)
        offsets(bufs[0])
        g_inflight = [None, None]
        out_inflight = [None, None]
        g_inflight[0] = gather_start(bufs[0], sems[0])
        idx_inflight = idx_start(1, bufs[1], isems[1])
        for c in range(1, nchunk):
            s = c % 2
            wait_all(idx_inflight)
            offsets(bufs[s])
            g_inflight[s] = gather_start(bufs[s], sems[s])
            if c + 1 < nchunk:
                idx_inflight = idx_start(c + 1, bufs[1 - s], isems[1 - s])
            wait_all(g_inflight[1 - s])
            if out_inflight[1 - s] is not None:
                out_inflight[1 - s].wait()
            out_inflight[1 - s] = combine(c - 1, bufs[1 - s], osems[1 - s])
        s = (nchunk - 1) % 2
        wait_all(g_inflight[s])
        if out_inflight[s] is not None:
            out_inflight[s].wait()
        combine(nchunk - 1, bufs[s], osems[s]).wait()
        if out_inflight[1 - s] is not None:
            out_inflight[1 - s].wait()

    return k(cols_hbm, a_flat)


_PARTS = (1000000,)


def kernel(triangles_indexes, A_s, triangles):
    del triangles
    tidx = triangles_indexes.astype(jnp.int32)
    table = A_s.reshape(512, 8, 32, 128).transpose(0, 2, 1, 3).reshape(-1)
    outs = []
    lo = 0
    for h in _PARTS:
        cols = tidx[lo:lo + h].T.reshape(-1)
        outs.append(_p_init_sc(cols, table, h))
        lo += h
    return jnp.concatenate(outs)


# speedup vs baseline: 1.1309x; 1.0010x over previous
import functools

import jax
import jax.numpy as jnp
from jax import lax
from jax.experimental import pallas as pl
from jax.experimental.pallas import tpu as pltpu
from jax.experimental.pallas import tpu_sc as plsc

_T = 1000000
_N = 4096
_L = 16
_C = 4096
_R = _C // 128
_NW = 32


def _p_init_sc(cols_hbm, a_flat, t):
    per_w = (t // _NW) // 16 * 16
    last_span = t - (_NW - 1) * per_w
    nchunk = -(-max(per_w, last_span) // _C)
    mesh = plsc.VectorSubcoreMesh(core_axis_name="c", subcore_axis_name="s")

    buf_set = [
        pltpu.VMEM((_C,), jnp.int32),
        pltpu.VMEM((_C,), jnp.int32),
        pltpu.VMEM((_C,), jnp.int32),
        pltpu.VMEM((3 * _C,), jnp.int32),
        pltpu.VMEM((3 * _C,), jnp.float32),
        pltpu.VMEM((_C,), jnp.float32),
    ]

    @functools.partial(
        pl.kernel,
        out_type=jax.ShapeDtypeStruct((t,), jnp.float32),
        mesh=mesh,
        scratch_types=buf_set + buf_set + [
            pltpu.SemaphoreType.DMA,
            pltpu.SemaphoreType.DMA,
            pltpu.SemaphoreType.DMA,
            pltpu.SemaphoreType.DMA,
            pltpu.SemaphoreType.DMA,
            pltpu.SemaphoreType.DMA,
        ],
        compiler_params=pltpu.CompilerParams(needs_layout_passes=False),
    )
    def k(ch, a_hbm, out_hbm, *scratch):
        bufs = (scratch[0:6], scratch[6:12])
        sems = scratch[12:14]
        isems = scratch[14:16]
        osems = scratch[16:18]
        nc = 2
        wid = lax.axis_index("s") * nc + lax.axis_index("c")
        start = wid * per_w
        end = jnp.where(wid == _NW - 1, t, start + per_w)
        lastbase = end - _C

        def chunk_base(c):
            return jnp.minimum(start + c * _C, lastbase)

        def idx_start(c, bset, sem):
            bi, bj, bk = bset[0:3]
            base = chunk_base(c)
            return [
                pltpu.async_copy(ch.at[pl.ds(col * t + base, _C)], dst, sem)
                for col, dst in ((0, bi), (1, bj), (2, bk))
            ]

        def offsets(bset):
            bi, bj, bk, f = bset[0:4]

            def fbody(r, carry):
                for g in range(8):
                    off = r * 128 + g * 16
                    iv = bi[pl.ds(off, 16)]
                    jv = bj[pl.ds(off, 16)]
                    kv = bk[pl.ds(off, 16)]
                    rpi = lax.shift_left(iv & 0xFF8, 12) | lax.shift_left(iv & 7, 7)
                    rpj = lax.shift_left(jv & 0xFF8, 12) | lax.shift_left(jv & 7, 7)
                    cpj = lax.shift_left(jv & 0xF80, 3) | (jv & 0x7F)
                    cpk = lax.shift_left(kv & 0xF80, 3) | (kv & 0x7F)
                    f[pl.ds(off, 16)] = rpi + cpj
                    f[pl.ds(_C + off, 16)] = rpi + cpk
                    f[pl.ds(2 * _C + off, 16)] = rpj + cpk
                return carry

            lax.fori_loop(0, _R, fbody, 0)

        def gather_start(bset, sem):
            f, g = bset[3:5]
            return [pltpu.async_copy(a_hbm.at[f], g, sem)]

        def combine(c, bset, sem):
            g, obuf = bset[4:6]
            base = chunk_base(c)

            def obody(r, carry):
                for gi in range(8):
                    off = r * 128 + gi * 16
                    s = (g[pl.ds(off, 16)]
                         + g[pl.ds(_C + off, 16)]
                         + g[pl.ds(2 * _C + off, 16)])
                    obuf[pl.ds(off, 16)] = s * (1.0 / 3.0)
                return carry

            lax.fori_loop(0, _R, obody, 0)
            return pltpu.async_copy(obuf, out_hbm.at[pl.ds(base, _C)], sem)

        def wait_all(copies):
            for cp in copies:
                cp.wait()

        idx_inflight = idx_start(0, bufs[0], isems[0])
        wait_all(idx_inflight)
        offsets(bufs[0])
        g_inflight = [None, None]
        out_inflight = [None, None]
        g_inflight[0] = gather_start(bufs[0], sems[0])
        idx_inflight = idx_start(1, bufs[1], isems[1])
        for c in range(1, nchunk):
            s = c % 2
            wait_all(idx_inflight)
            offsets(bufs[s])
            g_inflight[s] = gather_start(bufs[s], sems[s])
            if c + 1 < nchunk:
                idx_inflight = idx_start(c + 1, bufs[1 - s], isems[1 - s])
            wait_all(g_inflight[1 - s])
            if out_inflight[1 - s] is not None:
                out_inflight[1 - s].wait()
            out_inflight[1 - s] = combine(c - 1, bufs[1 - s], osems[1 - s])
        s = (nchunk - 1) % 2
        wait_all(g_inflight[s])
        if out_inflight[s] is not None:
            out_inflight[s].wait()
        combine(nchunk - 1, bufs[s], osems[s]).wait()
        if out_inflight[1 - s] is not None:
            out_inflight[1 - s].wait()

    return k(cols_hbm, a_flat)


_PARTS = (1000000,)


def kernel(triangles_indexes, A_s, triangles):
    del triangles
    tidx = triangles_indexes.astype(jnp.int32)
    table = A_s.reshape(512, 8, 32, 128).transpose(0, 2, 1, 3).reshape(-1)
    outs = []
    lo = 0
    for h in _PARTS:
        cols = tidx[lo:lo + h].T.reshape(-1)
        outs.append(_p_init_sc(cols, table, h))
        lo += h
    return outs[0] if len(outs) == 1 else jnp.concatenate(outs)
